# baseline (speedup 1.0000x reference)
import functools

import jax
import jax.numpy as jnp
from jax import lax
from jax.experimental import pallas as pl
from jax.experimental.pallas import tpu as pltpu
from jax.experimental.pallas import tpu_sc as plsc

N_AGENTS = 100000
N_VENUES = 2000
N_EDGES = 1600000
TAU = 0.1

NC = 2
NS = 16
NW = NC * NS
CHUNK = 2000
KC = N_EDGES // (NW * CHUNK)
NSLOT = 4
NA_PAD = 100096
ASLICE = NA_PAD // NS
NV_PAD = 2016
NVG = NV_PAD // 16

_MESH = plsc.VectorSubcoreMesh(
    core_axis_name="c", subcore_axis_name="s", num_cores=NC, num_subcores=NS)


def _slot_scratch():
    return (
        [pltpu.VMEM((CHUNK,), jnp.int32) for _ in range(NSLOT)] +
        [pltpu.VMEM((CHUNK,), jnp.int32) for _ in range(NSLOT)] +
        [pltpu.VMEM((CHUNK,), jnp.float32) for _ in range(NSLOT)] +
        [pltpu.SemaphoreType.DMA for _ in range(NSLOT)] +
        [pltpu.SemaphoreType.DMA for _ in range(NSLOT)] +
        [pltpu.SemaphoreType.DMA for _ in range(NSLOT)] +
        [pltpu.SemaphoreType.DMA for _ in range(NSLOT)]
    )


@functools.partial(
    pl.kernel,
    out_type=(
        jax.ShapeDtypeStruct((NC * NV_PAD,), jnp.float32),
        jax.ShapeDtypeStruct((NC * NV_PAD,), jnp.float32),
    ),
    mesh=_MESH,
    scratch_types=[
        pltpu.VMEM_SHARED((NA_PAD,), jnp.float32),
        pltpu.VMEM_SHARED((NV_PAD,), jnp.float32),
        pltpu.VMEM_SHARED((NV_PAD,), jnp.float32),
        pltpu.VMEM((ASLICE,), jnp.float32),
        pltpu.VMEM((ASLICE,), jnp.float32),
        pltpu.VMEM((NV_PAD,), jnp.float32),
        pltpu.VMEM((CHUNK,), jnp.float32),
    ] + _slot_scratch(),
)
def _venue_pass(tb_hbm, inf_hbm, ea_hbm, ev_hbm, vsum_out, vcnt_out,
                trans_sp, vsum_sp, vcnt_sp, tb_v, inf_v, zv, ones, *slots):
    cid = lax.axis_index("c")
    sid = lax.axis_index("s")
    wid = sid * NC + cid
    base = sid * ASLICE
    aidx = slots[0:NSLOT]
    vidx = slots[NSLOT:2 * NSLOT]
    vals = slots[2 * NSLOT:3 * NSLOT]
    dsem = slots[3 * NSLOT:4 * NSLOT]
    gsem = slots[4 * NSLOT:5 * NSLOT]
    s1sem = slots[5 * NSLOT:6 * NSLOT]
    s2sem = slots[6 * NSLOT:7 * NSLOT]

    pltpu.sync_copy(tb_hbm.at[pl.ds(base, ASLICE)], tb_v)
    pltpu.sync_copy(inf_hbm.at[pl.ds(base, ASLICE)], inf_v)

    def mul_body(i, carry):
        sl = pl.ds(i * 16, 16)
        tb_v[sl] = tb_v[sl] * inf_v[sl]
        return carry
    lax.fori_loop(0, ASLICE // 16, mul_body, 0)
    pltpu.sync_copy(tb_v, trans_sp.at[pl.ds(base, ASLICE)])

    def ones_body(i, carry):
        ones[pl.ds(i * 16, 16)] = jnp.full((16,), 1.0, jnp.float32)
        return carry
    lax.fori_loop(0, CHUNK // 16, ones_body, 0)

    @pl.when(sid == 0)
    def _():
        def zero_body(i, carry):
            zv[pl.ds(i * 16, 16)] = jnp.zeros((16,), jnp.float32)
            return carry
        lax.fori_loop(0, NVG, zero_body, 0)
        pltpu.sync_copy(zv, vsum_sp)
        pltpu.sync_copy(zv, vcnt_sp)

    def dma_in(k):
        s = k % NSLOT
        b = (k * NW + wid) * CHUNK
        da = pltpu.async_copy(ea_hbm.at[pl.ds(b, CHUNK)], aidx[s], dsem[s])
        dv = pltpu.async_copy(ev_hbm.at[pl.ds(b, CHUNK)], vidx[s], dsem[s])
        return (da, dv)

    def gather(k):
        s = k % NSLOT
        return pltpu.async_copy(trans_sp.at[aidx[s]], vals[s], gsem[s])

    def scat_cnt(k):
        s = k % NSLOT
        return pltpu.async_copy(ones, vcnt_sp.at[vidx[s]], s2sem[s], add=True)

    def scat_sum(k):
        s = k % NSLOT
        return pltpu.async_copy(vals[s], vsum_sp.at[vidx[s]], s1sem[s],
                                add=True)

    dmas = {0: dma_in(0), 1: dma_in(1)}
    gathers = {}
    scats = {}
    plsc.subcore_barrier()
    dmas[0][0].wait()
    dmas[0][1].wait()
    gathers[0] = gather(0)
    for k in range(KC):
        if k >= 2:
            scats[k - 2][0].wait()
            scats[k - 2][1].wait()
        d2 = scat_cnt(k)
        if k + 2 < KC:
            dmas[k + 2] = dma_in(k + 2)
        if k + 1 < KC:
            dmas[k + 1][0].wait()
            dmas[k + 1][1].wait()
            gathers[k + 1] = gather(k + 1)
        gathers[k].wait()
        scats[k] = (scat_sum(k), d2)
    for k in (KC - 2, KC - 1):
        scats[k][0].wait()
        scats[k][1].wait()
    plsc.subcore_barrier()

    @pl.when(sid == 0)
    def _():
        pltpu.sync_copy(vsum_sp, zv)
        pltpu.sync_copy(zv, vsum_out.at[pl.ds(cid * NV_PAD, NV_PAD)])
        pltpu.sync_copy(vcnt_sp, zv)
        pltpu.sync_copy(zv, vcnt_out.at[pl.ds(cid * NV_PAD, NV_PAD)])


@functools.partial(
    pl.kernel,
    out_type=jax.ShapeDtypeStruct((NC * NA_PAD,), jnp.float32),
    mesh=_MESH,
    scratch_types=[
        pltpu.VMEM_SHARED((NV_PAD,), jnp.float32),
        pltpu.VMEM_SHARED((NA_PAD,), jnp.float32),
        pltpu.VMEM((NC * NV_PAD,), jnp.float32),
        pltpu.VMEM((NC * NV_PAD,), jnp.float32),
        pltpu.VMEM((NV_PAD,), jnp.float32),
        pltpu.VMEM((ASLICE,), jnp.float32),
    ] + _slot_scratch(),
)
def _agent_pass(vsum_hbm, vcnt_hbm, beta_hbm, ea_hbm, ev_hbm, apart_out,
                fac_sp, ag_sp, vs, vc, bb, z, *slots):
    cid = lax.axis_index("c")
    sid = lax.axis_index("s")
    wid = sid * NC + cid
    base = sid * ASLICE
    aidx = slots[0:NSLOT]
    vidx = slots[NSLOT:2 * NSLOT]
    vals = slots[2 * NSLOT:3 * NSLOT]
    dsem = slots[3 * NSLOT:4 * NSLOT]
    gsem = slots[4 * NSLOT:5 * NSLOT]
    s1sem = slots[5 * NSLOT:6 * NSLOT]

    def zero_body(i, carry):
        z[pl.ds(i * 16, 16)] = jnp.zeros((16,), jnp.float32)
        return carry
    lax.fori_loop(0, ASLICE // 16, zero_body, 0)
    pltpu.sync_copy(z, ag_sp.at[pl.ds(base, ASLICE)])

    @pl.when(sid == 0)
    def _():
        pltpu.sync_copy(vsum_hbm, vs)
        pltpu.sync_copy(vcnt_hbm, vc)
        pltpu.sync_copy(beta_hbm, bb)

        def fac_body(i, carry):
            sl = pl.ds(i * 16, 16)
            sl1 = pl.ds(NV_PAD + i * 16, 16)
            s = vs[sl] + vs[sl1]
            cnt = vc[sl] + vc[sl1]
            bb[sl] = bb[sl] * s / jnp.maximum(cnt, 1.0)
            return carry
        lax.fori_loop(0, NVG, fac_body, 0)
        pltpu.sync_copy(bb, fac_sp)

    def dma_in(k):
        s = k % NSLOT
        b = (k * NW + wid) * CHUNK
        da = pltpu.async_copy(ea_hbm.at[pl.ds(b, CHUNK)], aidx[s], dsem[s])
        dv = pltpu.async_copy(ev_hbm.at[pl.ds(b, CHUNK)], vidx[s], dsem[s])
        return (da, dv)

    def gather(k):
        s = k % NSLOT
        return pltpu.async_copy(fac_sp.at[vidx[s]], vals[s], gsem[s])

    def scat(k):
        s = k % NSLOT
        return pltpu.async_copy(vals[s], ag_sp.at[aidx[s]], s1sem[s], add=True)

    dmas = {0: dma_in(0), 1: dma_in(1)}
    gathers = {}
    scats = {}
    plsc.subcore_barrier()
    dmas[0][0].wait()
    dmas[0][1].wait()
    gathers[0] = gather(0)
    for k in range(KC):
        if k >= 2:
            scats[k - 2].wait()
        if k + 2 < KC:
            dmas[k + 2] = dma_in(k + 2)
        if k + 1 < KC:
            dmas[k + 1][0].wait()
            dmas[k + 1][1].wait()
            gathers[k + 1] = gather(k + 1)
        gathers[k].wait()
        scats[k] = scat(k)
    scats[KC - 2].wait()
    scats[KC - 1].wait()

    plsc.subcore_barrier()
    pltpu.sync_copy(ag_sp.at[pl.ds(base, ASLICE)], z)
    pltpu.sync_copy(z, apart_out.at[pl.ds(cid * NA_PAD + base, ASLICE)])


def _elem_body(p0_ref, p1_ref, susc_ref, inf_ref, it_ref, u_ref,
               tn_ref, dt_ref, o_susc, o_inf, o_it, o_sym, o_pn):
    expo = p0_ref[...] + p1_ref[...]
    susc = susc_ref[...]
    inf = inf_ref[...]
    it = it_ref[...]
    u = u_ref[...]
    tn = tn_ref[0]
    dt = dt_ref[0]

    pn = jnp.exp(-dt * susc * expo)
    p = jnp.clip(pn, 1e-6, 1.0 - 1e-6)
    l0 = jnp.log(p)
    l1 = jnp.log1p(-p)
    g0 = -jnp.log(-jnp.log(u))
    g1 = -jnp.log(-jnp.log(1.0 - u))
    d = ((l1 + g1) - (l0 + g0)) / TAU
    y = 1.0 / (1.0 + jnp.exp(-d))

    inf_new = inf + y
    it_new = jnp.where(y > 0.5, tn, it)
    o_susc[...] = jnp.maximum(0.0, susc - y)
    o_inf[...] = inf_new
    o_it[...] = it_new
    o_sym[...] = inf_new * jnp.exp(-(tn - it_new))
    o_pn[...] = pn


def kernel(susceptibility, is_infected, infection_time, transmission_base,
           edge_agent, edge_venue, beta, noise_u, timer_now, delta_time):
    tb = jnp.pad(transmission_base, (0, NA_PAD - N_AGENTS))
    ip = jnp.pad(is_infected, (0, NA_PAD - N_AGENTS))
    bp = jnp.pad(beta, (0, NV_PAD - N_VENUES))

    vsum, vcnt = _venue_pass(tb, ip, edge_agent, edge_venue)
    apart = _agent_pass(vsum, vcnt, bp, edge_agent, edge_venue)
    p0 = apart[:N_AGENTS]
    p1 = apart[NA_PAD:NA_PAD + N_AGENTS]

    vec = jax.ShapeDtypeStruct((N_AGENTS,), jnp.float32)
    smem_spec = pl.BlockSpec(memory_space=pltpu.SMEM)
    return pl.pallas_call(
        _elem_body,
        out_shape=(vec, vec, vec, vec, vec),
        in_specs=[pl.BlockSpec()] * 6 + [smem_spec, smem_spec],
    )(p0, p1, susceptibility, is_infected, infection_time, noise_u,
      timer_now, delta_time)


# speedup vs baseline: 1.0398x; 1.0398x over previous
import functools

import jax
import jax.numpy as jnp
from jax import lax
from jax.experimental import pallas as pl
from jax.experimental.pallas import tpu as pltpu
from jax.experimental.pallas import tpu_sc as plsc

N_AGENTS = 100000
N_VENUES = 2000
N_EDGES = 1600000
TAU = 0.1

NC = 2
NS = 16
NW = NC * NS
CHUNK = 2000
KC = N_EDGES // (NW * CHUNK)
NSLOT = 4
NA_PAD = 100096
ASLICE = NA_PAD // NS
NV_PAD = 2016
NVG = NV_PAD // 16

_MESH = plsc.VectorSubcoreMesh(
    core_axis_name="c", subcore_axis_name="s", num_cores=NC, num_subcores=NS)


def _slot_scratch():
    return (
        [pltpu.VMEM((CHUNK,), jnp.int32) for _ in range(NSLOT)] +
        [pltpu.VMEM((CHUNK,), jnp.int32) for _ in range(NSLOT)] +
        [pltpu.VMEM((CHUNK,), jnp.float32) for _ in range(NSLOT)] +
        [pltpu.SemaphoreType.DMA for _ in range(NSLOT)] +
        [pltpu.SemaphoreType.DMA for _ in range(NSLOT)] +
        [pltpu.SemaphoreType.DMA for _ in range(NSLOT)] +
        [pltpu.SemaphoreType.DMA for _ in range(NSLOT)]
    )


@functools.partial(
    pl.kernel,
    out_type=(
        jax.ShapeDtypeStruct((NC * NV_PAD,), jnp.float32),
        jax.ShapeDtypeStruct((NC * NV_PAD,), jnp.float32),
    ),
    mesh=_MESH,
    scratch_types=[
        pltpu.VMEM_SHARED((NA_PAD,), jnp.float32),
        pltpu.VMEM_SHARED((NV_PAD,), jnp.float32),
        pltpu.VMEM_SHARED((NV_PAD,), jnp.float32),
        pltpu.VMEM((ASLICE,), jnp.float32),
        pltpu.VMEM((ASLICE,), jnp.float32),
        pltpu.VMEM((NV_PAD,), jnp.float32),
        pltpu.VMEM((CHUNK,), jnp.float32),
    ] + _slot_scratch(),
)
def _venue_pass(tb_hbm, inf_hbm, ea_hbm, ev_hbm, vsum_out, vcnt_out,
                trans_sp, vsum_sp, vcnt_sp, tb_v, inf_v, zv, ones, *slots):
    cid = lax.axis_index("c")
    sid = lax.axis_index("s")
    wid = sid * NC + cid
    base = sid * ASLICE
    aidx = slots[0:NSLOT]
    vidx = slots[NSLOT:2 * NSLOT]
    vals = slots[2 * NSLOT:3 * NSLOT]
    dsem = slots[3 * NSLOT:4 * NSLOT]
    gsem = slots[4 * NSLOT:5 * NSLOT]
    s1sem = slots[5 * NSLOT:6 * NSLOT]
    s2sem = slots[6 * NSLOT:7 * NSLOT]

    pltpu.sync_copy(tb_hbm.at[pl.ds(base, ASLICE)], tb_v)
    pltpu.sync_copy(inf_hbm.at[pl.ds(base, ASLICE)], inf_v)

    def mul_body(i, carry):
        sl = pl.ds(i * 16, 16)
        tb_v[sl] = tb_v[sl] * inf_v[sl]
        return carry
    lax.fori_loop(0, ASLICE // 16, mul_body, 0)
    pltpu.sync_copy(tb_v, trans_sp.at[pl.ds(base, ASLICE)])

    def ones_body(i, carry):
        ones[pl.ds(i * 16, 16)] = jnp.full((16,), 1.0, jnp.float32)
        return carry
    lax.fori_loop(0, CHUNK // 16, ones_body, 0)

    @pl.when(sid == 0)
    def _():
        def zero_body(i, carry):
            zv[pl.ds(i * 16, 16)] = jnp.zeros((16,), jnp.float32)
            return carry
        lax.fori_loop(0, NVG, zero_body, 0)
        pltpu.sync_copy(zv, vsum_sp)
        pltpu.sync_copy(zv, vcnt_sp)
    plsc.subcore_barrier()

    def dma_in(k):
        s = k % NSLOT
        b = (k * NW + wid) * CHUNK
        da = pltpu.async_copy(ea_hbm.at[pl.ds(b, CHUNK)], aidx[s], dsem[s])
        dv = pltpu.async_copy(ev_hbm.at[pl.ds(b, CHUNK)], vidx[s], dsem[s])
        return (da, dv)

    def gather(k):
        s = k % NSLOT
        return pltpu.async_copy(trans_sp.at[aidx[s]], vals[s], gsem[s])

    def scat_cnt(k):
        s = k % NSLOT
        return pltpu.async_copy(ones, vcnt_sp.at[vidx[s]], s2sem[s], add=True)

    def scat_sum(k):
        s = k % NSLOT
        return pltpu.async_copy(vals[s], vsum_sp.at[vidx[s]], s1sem[s],
                                add=True)

    dmas = {0: dma_in(0), 1: dma_in(1)}
    gathers = {}
    scats = {}
    dmas[0][0].wait()
    dmas[0][1].wait()
    gathers[0] = gather(0)
    for k in range(KC):
        if k >= 2:
            scats[k - 2][0].wait()
            scats[k - 2][1].wait()
        d2 = scat_cnt(k)
        if k + 2 < KC:
            dmas[k + 2] = dma_in(k + 2)
        if k + 1 < KC:
            dmas[k + 1][0].wait()
            dmas[k + 1][1].wait()
            gathers[k + 1] = gather(k + 1)
        gathers[k].wait()
        scats[k] = (scat_sum(k), d2)
    for k in (KC - 2, KC - 1):
        scats[k][0].wait()
        scats[k][1].wait()
    plsc.subcore_barrier()

    @pl.when(sid == 0)
    def _():
        pltpu.sync_copy(vsum_sp, zv)
        pltpu.sync_copy(zv, vsum_out.at[pl.ds(cid * NV_PAD, NV_PAD)])
        pltpu.sync_copy(vcnt_sp, zv)
        pltpu.sync_copy(zv, vcnt_out.at[pl.ds(cid * NV_PAD, NV_PAD)])


@functools.partial(
    pl.kernel,
    out_type=jax.ShapeDtypeStruct((NC * NA_PAD,), jnp.float32),
    mesh=_MESH,
    scratch_types=[
        pltpu.VMEM_SHARED((NV_PAD,), jnp.float32),
        pltpu.VMEM_SHARED((NA_PAD,), jnp.float32),
        pltpu.VMEM((NC * NV_PAD,), jnp.float32),
        pltpu.VMEM((NC * NV_PAD,), jnp.float32),
        pltpu.VMEM((NV_PAD,), jnp.float32),
        pltpu.VMEM((ASLICE,), jnp.float32),
    ] + _slot_scratch(),
)
def _agent_pass(vsum_hbm, vcnt_hbm, beta_hbm, ea_hbm, ev_hbm, apart_out,
                fac_sp, ag_sp, vs, vc, bb, z, *slots):
    cid = lax.axis_index("c")
    sid = lax.axis_index("s")
    wid = sid * NC + cid
    base = sid * ASLICE
    aidx = slots[0:NSLOT]
    vidx = slots[NSLOT:2 * NSLOT]
    vals = slots[2 * NSLOT:3 * NSLOT]
    dsem = slots[3 * NSLOT:4 * NSLOT]
    gsem = slots[4 * NSLOT:5 * NSLOT]
    s1sem = slots[5 * NSLOT:6 * NSLOT]

    def zero_body(i, carry):
        z[pl.ds(i * 16, 16)] = jnp.zeros((16,), jnp.float32)
        return carry
    lax.fori_loop(0, ASLICE // 16, zero_body, 0)
    pltpu.sync_copy(z, ag_sp.at[pl.ds(base, ASLICE)])

    @pl.when(sid == 0)
    def _():
        pltpu.sync_copy(vsum_hbm, vs)
        pltpu.sync_copy(vcnt_hbm, vc)
        pltpu.sync_copy(beta_hbm, bb)

        def fac_body(i, carry):
            sl = pl.ds(i * 16, 16)
            sl1 = pl.ds(NV_PAD + i * 16, 16)
            s = vs[sl] + vs[sl1]
            cnt = vc[sl] + vc[sl1]
            bb[sl] = bb[sl] * s / jnp.maximum(cnt, 1.0)
            return carry
        lax.fori_loop(0, NVG, fac_body, 0)
        pltpu.sync_copy(bb, fac_sp)
    plsc.subcore_barrier()

    def dma_in(k):
        s = k % NSLOT
        b = (k * NW + wid) * CHUNK
        da = pltpu.async_copy(ea_hbm.at[pl.ds(b, CHUNK)], aidx[s], dsem[s])
        dv = pltpu.async_copy(ev_hbm.at[pl.ds(b, CHUNK)], vidx[s], dsem[s])
        return (da, dv)

    def gather(k):
        s = k % NSLOT
        return pltpu.async_copy(fac_sp.at[vidx[s]], vals[s], gsem[s])

    def scat(k):
        s = k % NSLOT
        return pltpu.async_copy(vals[s], ag_sp.at[aidx[s]], s1sem[s], add=True)

    dmas = {0: dma_in(0), 1: dma_in(1)}
    gathers = {}
    scats = {}
    dmas[0][0].wait()
    dmas[0][1].wait()
    gathers[0] = gather(0)
    for k in range(KC):
        if k >= 2:
            scats[k - 2].wait()
        if k + 2 < KC:
            dmas[k + 2] = dma_in(k + 2)
        if k + 1 < KC:
            dmas[k + 1][0].wait()
            dmas[k + 1][1].wait()
            gathers[k + 1] = gather(k + 1)
        gathers[k].wait()
        scats[k] = scat(k)
    scats[KC - 2].wait()
    scats[KC - 1].wait()

    plsc.subcore_barrier()
    pltpu.sync_copy(ag_sp.at[pl.ds(base, ASLICE)], z)
    pltpu.sync_copy(z, apart_out.at[pl.ds(cid * NA_PAD + base, ASLICE)])


def _elem_body(p0_ref, p1_ref, susc_ref, inf_ref, it_ref, u_ref,
               tn_ref, dt_ref, o_susc, o_inf, o_it, o_sym, o_pn):
    expo = p0_ref[...] + p1_ref[...]
    susc = susc_ref[...]
    inf = inf_ref[...]
    it = it_ref[...]
    u = u_ref[...]
    tn = tn_ref[0]
    dt = dt_ref[0]

    pn = jnp.exp(-dt * susc * expo)
    p = jnp.clip(pn, 1e-6, 1.0 - 1e-6)
    l0 = jnp.log(p)
    l1 = jnp.log1p(-p)
    g0 = -jnp.log(-jnp.log(u))
    g1 = -jnp.log(-jnp.log(1.0 - u))
    d = ((l1 + g1) - (l0 + g0)) / TAU
    y = 1.0 / (1.0 + jnp.exp(-d))

    inf_new = inf + y
    it_new = jnp.where(y > 0.5, tn, it)
    o_susc[...] = jnp.maximum(0.0, susc - y)
    o_inf[...] = inf_new
    o_it[...] = it_new
    o_sym[...] = inf_new * jnp.exp(-(tn - it_new))
    o_pn[...] = pn


def kernel(susceptibility, is_infected, infection_time, transmission_base,
           edge_agent, edge_venue, beta, noise_u, timer_now, delta_time):
    tb = jnp.pad(transmission_base, (0, NA_PAD - N_AGENTS))
    ip = jnp.pad(is_infected, (0, NA_PAD - N_AGENTS))
    bp = jnp.pad(beta, (0, NV_PAD - N_VENUES))

    vsum, vcnt = _venue_pass(tb, ip, edge_agent, edge_venue)
    apart = _agent_pass(vsum, vcnt, bp, edge_agent, edge_venue)
    p0 = apart[:N_AGENTS]
    p1 = apart[NA_PAD:NA_PAD + N_AGENTS]

    vec = jax.ShapeDtypeStruct((N_AGENTS,), jnp.float32)
    smem_spec = pl.BlockSpec(memory_space=pltpu.SMEM)
    return pl.pallas_call(
        _elem_body,
        out_shape=(vec, vec, vec, vec, vec),
        in_specs=[pl.BlockSpec()] * 6 + [smem_spec, smem_spec],
    )(p0, p1, susceptibility, is_infected, infection_time, noise_u,
      timer_now, delta_time)
